# initial kernel scaffold (unmeasured)
import jax
import jax.numpy as jnp
from jax import lax
from jax.experimental import pallas as pl
from jax.experimental.pallas import tpu as pltpu

NY = 4
T = 2048
D = 1024
MAXC = 576


def kernel(x, dest):
    def body(xs_ref, cntv_ref, cnts_ref, l_ref, out_ref,
             stag_ref, pout_ref, cstage_ref, csmem_ref,
             dsend, drecv, csend, crecv, lsem):
        my_x = lax.axis_index("x")
        my_y = lax.axis_index("y")
        my_z = lax.axis_index("z")

        def peer(d):
            return (my_x, d, my_z)

        bsem = pltpu.get_barrier_semaphore()
        for d in range(NY):
            @pl.when(d != my_y)
            def _():
                pl.semaphore_signal(
                    bsem, inc=1, device_id=peer(d),
                    device_id_type=pl.DeviceIdType.MESH)
        pl.semaphore_wait(bsem, NY - 1)

        def data_rdma(d):
            return pltpu.make_async_remote_copy(
                src_ref=xs_ref.at[pl.ds(l_ref[d], MAXC)],
                dst_ref=stag_ref.at[my_y],
                send_sem=dsend.at[d],
                recv_sem=drecv.at[my_y],
                device_id=peer(d),
                device_id_type=pl.DeviceIdType.MESH)

        def cnt_rdma(d):
            return pltpu.make_async_remote_copy(
                src_ref=cntv_ref.at[pl.ds(d, 1)],
                dst_ref=cstage_ref.at[pl.ds(my_y, 1)],
                send_sem=csend.at[d],
                recv_sem=crecv.at[my_y],
                device_id=peer(d),
                device_id_type=pl.DeviceIdType.MESH)

        for d in range(NY):
            @pl.when(d != my_y)
            def _():
                cnt_rdma(d).start()
                data_rdma(d).start()

        for s in range(NY):
            @pl.when(s != my_y)
            def _():
                pltpu.make_async_remote_copy(
                    src_ref=cntv_ref.at[pl.ds(s, 1)],
                    dst_ref=cstage_ref.at[pl.ds(s, 1)],
                    send_sem=csend.at[s],
                    recv_sem=crecv.at[s],
                    device_id=peer(s),
                    device_id_type=pl.DeviceIdType.MESH).wait_recv()

        cp = pltpu.make_async_copy(cstage_ref, csmem_ref, lsem)
        cp.start()
        cp.wait()

        r = jnp.int32(0)
        for s in range(NY):
            c_s = jnp.where(my_y == s, cnts_ref[my_y], csmem_ref[s, 0])
            r_now = r

            @pl.when(my_y == s)
            def _():
                own = pltpu.make_async_copy(
                    xs_ref.at[pl.ds(l_ref[my_y], MAXC)],
                    pout_ref.at[pl.ds(r_now, MAXC)],
                    lsem)
                own.start()
                own.wait()

            @pl.when(my_y != s)
            def _():
                pltpu.make_async_remote_copy(
                    src_ref=xs_ref.at[pl.ds(l_ref[s], MAXC)],
                    dst_ref=stag_ref.at[s],
                    send_sem=dsend.at[s],
                    recv_sem=drecv.at[s],
                    device_id=peer(s),
                    device_id_type=pl.DeviceIdType.MESH).wait_recv()
                mv = pltpu.make_async_copy(
                    stag_ref.at[s],
                    pout_ref.at[pl.ds(r_now, MAXC)],
                    lsem)
                mv.start()
                mv.wait()

            r = r + c_s

        fin = pltpu.make_async_copy(
            pout_ref.at[pl.ds(0, T)], out_ref, lsem)
        fin.start()
        fin.wait()

        for d in range(NY):
            @pl.when(d != my_y)
            def _():
                cnt_rdma(d).wait_send()
                data_rdma(d).wait_send()

    order = jnp.argsort(dest, stable=True)
    xs = jnp.take(x, order, axis=0)
    xs_pad = jnp.pad(xs, ((0, MAXC), (0, 0)))
    counts = jnp.bincount(dest, length=NY).astype(jnp.int32)
    starts = jnp.concatenate(
        [jnp.zeros((1,), jnp.int32), jnp.cumsum(counts)[:-1].astype(jnp.int32)])
    cnt_vmem = jnp.broadcast_to(counts[:, None], (NY, 128)).astype(jnp.int32)

    return pl.pallas_call(
        body,
        out_shape=jax.ShapeDtypeStruct((T, D), jnp.float32),
        in_specs=[
            pl.BlockSpec(memory_space=pltpu.VMEM),
            pl.BlockSpec(memory_space=pltpu.VMEM),
            pl.BlockSpec(memory_space=pltpu.SMEM),
            pl.BlockSpec(memory_space=pltpu.SMEM),
        ],
        out_specs=pl.BlockSpec(memory_space=pltpu.VMEM),
        scratch_shapes=[
            pltpu.VMEM((NY, MAXC, D), jnp.float32),
            pltpu.VMEM((T + MAXC, D), jnp.float32),
            pltpu.VMEM((NY, 128), jnp.int32),
            pltpu.SMEM((NY, 128), jnp.int32),
            pltpu.SemaphoreType.DMA((NY,)),
            pltpu.SemaphoreType.DMA((NY,)),
            pltpu.SemaphoreType.DMA((NY,)),
            pltpu.SemaphoreType.DMA((NY,)),
            pltpu.SemaphoreType.DMA,
        ],
        compiler_params=pltpu.CompilerParams(collective_id=0),
    )(xs_pad, cnt_vmem, counts, starts)


# baseline (device time: 594422 ns/iter reference)
import jax
import jax.numpy as jnp
from jax import lax
from jax.experimental import pallas as pl
from jax.experimental.pallas import tpu as pltpu

NY = 4
T = 2048
D = 1024
MAXC = 576


def _body(sendbuf_ref, cntv_ref, blocks_ref, cnt_out_ref,
          dsend, drecv, csend, crecv, lsem):
    my_x = lax.axis_index("x")
    my_y = lax.axis_index("y")
    my_z = lax.axis_index("z")

    def peer(d):
        return (my_x, d, my_z)

    bsem = pltpu.get_barrier_semaphore()
    for d in range(NY):
        @pl.when(d != my_y)
        def _():
            pl.semaphore_signal(
                bsem, inc=1, device_id=peer(d),
                device_id_type=pl.DeviceIdType.MESH)
    pl.semaphore_wait(bsem, NY - 1)

    def data_rdma(d, slot):
        return pltpu.make_async_remote_copy(
            src_ref=sendbuf_ref.at[d],
            dst_ref=blocks_ref.at[slot],
            send_sem=dsend.at[d],
            recv_sem=drecv.at[slot],
            device_id=peer(d),
            device_id_type=pl.DeviceIdType.MESH)

    def cnt_rdma(d, slot):
        return pltpu.make_async_remote_copy(
            src_ref=cntv_ref.at[d],
            dst_ref=cnt_out_ref.at[slot],
            send_sem=csend.at[d],
            recv_sem=crecv.at[slot],
            device_id=peer(d),
            device_id_type=pl.DeviceIdType.MESH)

    for d in range(NY):
        @pl.when(d != my_y)
        def _():
            cnt_rdma(d, my_y).start()
            data_rdma(d, my_y).start()

    own_c = pltpu.make_async_copy(
        cntv_ref.at[my_y], cnt_out_ref.at[my_y], lsem)
    own_c.start()
    own_c.wait()
    own_d = pltpu.make_async_copy(
        sendbuf_ref.at[my_y], blocks_ref.at[my_y], lsem)
    own_d.start()
    own_d.wait()

    for s in range(NY):
        @pl.when(s != my_y)
        def _():
            cnt_rdma(s, s).wait_recv()
            data_rdma(s, s).wait_recv()

    for d in range(NY):
        @pl.when(d != my_y)
        def _():
            cnt_rdma(d, my_y).wait_send()
            data_rdma(d, my_y).wait_send()


def kernel(x, dest):
    counts = jnp.bincount(dest, length=NY).astype(jnp.int32)
    starts = jnp.concatenate(
        [jnp.zeros((1,), jnp.int32), jnp.cumsum(counts)[:-1].astype(jnp.int32)])
    order = jnp.argsort(dest, stable=True)
    order_pad = jnp.pad(order, (0, MAXC))
    gidx = order_pad[starts[:, None] + jnp.arange(MAXC)[None, :]]
    sendbuf = x[gidx]
    cntv = jnp.broadcast_to(
        counts[:, None, None], (NY, 8, 128)).astype(jnp.int32)

    blocks, cnt_out = pl.pallas_call(
        _body,
        out_shape=[
            jax.ShapeDtypeStruct((NY, MAXC, D), jnp.float32),
            jax.ShapeDtypeStruct((NY, 8, 128), jnp.int32),
        ],
        in_specs=[
            pl.BlockSpec(memory_space=pltpu.VMEM),
            pl.BlockSpec(memory_space=pltpu.VMEM),
        ],
        out_specs=[
            pl.BlockSpec(memory_space=pltpu.VMEM),
            pl.BlockSpec(memory_space=pltpu.VMEM),
        ],
        scratch_shapes=[
            pltpu.SemaphoreType.DMA((NY,)),
            pltpu.SemaphoreType.DMA((NY,)),
            pltpu.SemaphoreType.DMA((NY,)),
            pltpu.SemaphoreType.DMA((NY,)),
            pltpu.SemaphoreType.DMA,
        ],
        compiler_params=pltpu.CompilerParams(collective_id=0),
    )(sendbuf, cntv)

    c_in = cnt_out[:, 0, 0]
    csum = jnp.cumsum(c_in)
    off = csum - c_in
    t = jnp.arange(T)
    sid = jnp.searchsorted(csum, t, side="right")
    rid = t - off[sid]
    return blocks.reshape(NY * MAXC, D)[sid * MAXC + rid]
